# baseline (speedup 1.0000x reference)
import jax
import jax.numpy as jnp
from jax.experimental import pallas as pl

_PACK = 4
_BN = 4096


def _mlp_body(x_ref, w1_ref, b1_ref, w2_ref, b2_ref, o_ref):
    x = x_ref[...]
    h = jnp.dot(x, w1_ref[...], preferred_element_type=jnp.float32)
    h = h + b1_ref[...]
    h = jnp.where(h >= 0, h, 0.01 * h)
    o = jnp.dot(h, w2_ref[...], preferred_element_type=jnp.float32)
    o_ref[...] = o + b2_ref[...]


def kernel(batch_x, W1, b1, W2, b2):
    n, d_in = batch_x.shape
    d_h = W1.shape[0]
    n_bins = W2.shape[0]
    p = _PACK

    eye = jnp.eye(p, dtype=batch_x.dtype)
    w1big = jnp.kron(eye, W1.T)
    w2big = jnp.kron(eye, W2.T)
    b1big = jnp.tile(b1, p).reshape(1, p * d_h)
    b2big = jnp.tile(b2, p).reshape(1, p * n_bins)

    xp = batch_x.reshape(n // p, p * d_in)

    grid = (n // p) // _BN
    outp = pl.pallas_call(
        _mlp_body,
        grid=(grid,),
        in_specs=[
            pl.BlockSpec((_BN, p * d_in), lambda i: (i, 0)),
            pl.BlockSpec((p * d_in, p * d_h), lambda i: (0, 0)),
            pl.BlockSpec((1, p * d_h), lambda i: (0, 0)),
            pl.BlockSpec((p * d_h, p * n_bins), lambda i: (0, 0)),
            pl.BlockSpec((1, p * n_bins), lambda i: (0, 0)),
        ],
        out_specs=pl.BlockSpec((_BN, p * n_bins), lambda i: (i, 0)),
        out_shape=jax.ShapeDtypeStruct((n // p, p * n_bins), jnp.float32),
    )(xp, w1big, b1big, w2big, b2big)
    return outp.reshape(n, n_bins)


# speedup vs baseline: 1.3267x; 1.3267x over previous
import jax
import jax.numpy as jnp
from jax.experimental import pallas as pl

_BN = 16384


def _mlp_body(x_ref, w1_ref, b1_ref, w2_ref, b2_ref, o_ref):
    x = x_ref[...]
    h = jnp.dot(x, w1_ref[...], preferred_element_type=jnp.float32)
    h = h + b1_ref[...]
    h = jnp.where(h >= 0, h, 0.01 * h)
    o = jnp.dot(h, w2_ref[...], preferred_element_type=jnp.float32)
    o_ref[...] = o + b2_ref[...]


def kernel(batch_x, W1, b1, W2, b2):
    n, d_in = batch_x.shape
    d_h = W1.shape[0]
    n_bins = W2.shape[0]

    w1t = W1.T
    w2t = W2.T
    b1r = b1.reshape(1, d_h)
    b2r = b2.reshape(1, n_bins)

    grid = n // _BN
    return pl.pallas_call(
        _mlp_body,
        grid=(grid,),
        in_specs=[
            pl.BlockSpec((_BN, d_in), lambda i: (i, 0)),
            pl.BlockSpec((d_in, d_h), lambda i: (0, 0)),
            pl.BlockSpec((1, d_h), lambda i: (0, 0)),
            pl.BlockSpec((d_h, n_bins), lambda i: (0, 0)),
            pl.BlockSpec((1, n_bins), lambda i: (0, 0)),
        ],
        out_specs=pl.BlockSpec((_BN, n_bins), lambda i: (i, 0)),
        out_shape=jax.ShapeDtypeStruct((n, n_bins), jnp.float32),
    )(batch_x, w1t, b1r, w2t, b2r)
